# baseline (speedup 1.0000x reference)
import functools

import jax
import jax.numpy as jnp
from jax import lax
from jax.experimental import pallas as pl
from jax.experimental.pallas import tpu as pltpu
from jax.experimental.pallas import tpu_sc as plsc

N = 262144
C = 100
TOT = N * C
NC = 2
NS = 16
NW = NC * NS
WPW = TOT // NW
R = 512
CW = R * C
NG = WPW // CW
PAT = 400
NPAT = PAT // 16


def _body(feat_hbm, pat_hbm, out_hbm, buf, fillv):
    wid = lax.axis_index("s") * NC + lax.axis_index("c")
    base = wid * WPW
    pltpu.sync_copy(pat_hbm, fillv)
    fills = [fillv[pl.ds(16 * p, 16)] for p in range(NPAT)]

    def grp(i, carry):
        b0 = i * PAT
        for p in range(NPAT):
            o = b0 + 16 * p
            x = buf[pl.ds(o, 16)]
            buf[pl.ds(o, 16)] = jnp.where(x != x, fills[p], x)
        return carry

    for g in range(NG):
        off = pl.multiple_of(base + g * CW, 8)
        pltpu.sync_copy(feat_hbm.at[pl.ds(off, CW)], buf)
        lax.fori_loop(0, CW // PAT, grp, 0)
        pltpu.sync_copy(buf, out_hbm.at[pl.ds(off, CW)])


@jax.jit
def _sc_fill(flat, pattern):
    mesh = plsc.VectorSubcoreMesh(core_axis_name="c", subcore_axis_name="s")
    fn = functools.partial(
        pl.kernel,
        mesh=mesh,
        out_type=jax.ShapeDtypeStruct((TOT,), jnp.float32),
        scratch_types=[
            pltpu.VMEM((CW,), jnp.float32),
            pltpu.VMEM((PAT,), jnp.float32),
        ],
    )(_body)
    return fn(flat, pattern)


def kernel(feat, fill_values):
    fv = jnp.where(jnp.isnan(fill_values), 0.0, fill_values)
    pattern = jnp.tile(fv, PAT // C)
    out = _sc_fill(feat.reshape(TOT), pattern)
    return out.reshape(feat.shape)
